# baseline (speedup 1.0000x reference)
import functools

import jax
import jax.numpy as jnp
from jax import lax
from jax.experimental import pallas as pl
from jax.experimental.pallas import tpu as pltpu
from jax.experimental.pallas import tpu_sc as plsc

E = 8
TOPK = 2
BLK = 256
_NC = 2
_NS = 16
_NW = _NC * _NS
_LANES = 16


def _mix_body(up_logits_ref, gate_logits_ref, up_bank_ref, gate_bank_ref,
              up_out_ref, gate_out_ref):
    for lref, bref, oref in ((up_logits_ref, up_bank_ref, up_out_ref),
                             (gate_logits_ref, gate_bank_ref, gate_out_ref)):
        logits = lref[...]
        m = jnp.max(logits, axis=1, keepdims=True)
        ex = jnp.exp(logits - m)
        alpha = ex / jnp.sum(ex, axis=1, keepdims=True)
        mixed = jax.lax.dot_general(
            alpha, bref[...], (((1,), (0,)), ((), ())),
            preferred_element_type=jnp.float32)
        oref[...] = mixed.astype(jnp.bfloat16)


def _mix_banks(up_mixture_logits, gate_mixture_logits, up_bank, gate_bank):
    ku, d, r = up_bank.shape
    kg = gate_bank.shape[0]
    up_flat = up_bank.reshape(ku, d * r)
    gate_flat = gate_bank.reshape(kg, d * r)
    up_mixed, gate_mixed = pl.pallas_call(
        _mix_body,
        out_shape=(jax.ShapeDtypeStruct((E, d * r), jnp.bfloat16),
                   jax.ShapeDtypeStruct((E, d * r), jnp.bfloat16)),
    )(up_mixture_logits, gate_mixture_logits, up_flat, gate_flat)
    return up_mixed.reshape(E, d, r), gate_mixed.reshape(E, d, r)


def _routing_metadata(expert_indices, expert_weights, nb):
    i32 = jnp.int32
    g = expert_indices.size
    flat_e = expert_indices.reshape(-1).astype(i32)
    flat_w = expert_weights.reshape(-1)
    oh = flat_e[:, None] == jnp.arange(E, dtype=i32)[None, :]
    ohf = oh.astype(i32)
    ranks_all = jnp.cumsum(ohf, axis=0) - ohf
    rank = jnp.sum(jnp.where(oh, ranks_all, 0), axis=1)
    counts = jnp.sum(ohf, axis=0)
    bpe = (counts + BLK - 1) // BLK
    cumb = jnp.cumsum(bpe)
    base_slot = (jnp.concatenate([jnp.zeros(1, i32), cumb[:-1]]) * BLK)
    slot = base_slot[flat_e] + rank
    used = cumb[-1]
    bids = jnp.arange(nb, dtype=i32)
    raw_be = jnp.searchsorted(cumb, bids, side="right").astype(i32)
    last_e = jnp.searchsorted(cumb, used - 1, side="right").astype(i32)
    be = jnp.minimum(raw_be, last_e)
    block_start = bids * BLK
    nv = jnp.clip(counts[be] - (block_start - base_slot[be]), 0, BLK)
    nv = nv.astype(i32)
    first = ((block_start == base_slot[be]) & (nv > 0)).astype(i32)
    return flat_w, slot, be, nv, first


def _dispatch_sc(slot, flat_t, flat_w, hidden):
    g = slot.shape[0]
    seq, d = hidden.shape
    nbb = (g // BLK + E) * BLK
    rpw = nbb // _NW
    chunk = 64
    mesh = plsc.VectorSubcoreMesh(core_axis_name="c", subcore_axis_name="s")

    @functools.partial(
        pl.kernel, mesh=mesh,
        out_type=(jax.ShapeDtypeStruct((nbb, d), jnp.float32),
                  jax.ShapeDtypeStruct((nbb,), jnp.float32)),
        scratch_types=[
            pltpu.VMEM((g,), jnp.int32),
            pltpu.VMEM((g,), jnp.int32),
            pltpu.VMEM((g,), jnp.float32),
            pltpu.VMEM((rpw,), jnp.int32),
            pltpu.VMEM((rpw,), jnp.float32),
            pltpu.VMEM((chunk, d), jnp.float32),
            pltpu.SemaphoreType.DMA,
        ],
        compiler_params=pltpu.CompilerParams(needs_layout_passes=False),
    )
    def k(slot_hbm, tok_hbm, w_hbm, hid_hbm, xs_hbm, sw_hbm,
          slot_v, tok_v, w_v, st_v, swl_v, rows_v, sem):
        wid = lax.axis_index("s") * _NC + lax.axis_index("c")
        base = wid * rpw
        pltpu.sync_copy(slot_hbm, slot_v)
        pltpu.sync_copy(tok_hbm, tok_v)
        pltpu.sync_copy(w_hbm, w_v)

        def zbody(i, carry):
            sl = pl.ds(i * _LANES, _LANES)
            st_v[sl] = jnp.zeros((_LANES,), jnp.int32)
            swl_v[sl] = jnp.zeros((_LANES,), jnp.float32)
            return carry

        lax.fori_loop(0, rpw // _LANES, zbody, 0)

        def sbody(i, carry):
            sl = pl.ds(i * _LANES, _LANES)
            rel = slot_v[sl] - base
            msk = (rel >= 0) & (rel < rpw)
            plsc.store_scatter(st_v, [rel], tok_v[sl], mask=msk)
            plsc.store_scatter(swl_v, [rel], w_v[sl], mask=msk)
            return carry

        lax.fori_loop(0, g // _LANES, sbody, 0)
        pltpu.sync_copy(swl_v, sw_hbm.at[pl.ds(base, rpw)])
        for j in range(rpw // chunk):
            idx = st_v.at[pl.ds(j * chunk, chunk)]
            pltpu.async_copy(hid_hbm.at[idx], rows_v, sem).wait()
            pltpu.sync_copy(rows_v, xs_hbm.at[pl.ds(base + j * chunk, chunk)])

    return k(slot, flat_t, flat_w, hidden)


def _combine_sc(pos0, pos1, y_sorted):
    nrow, tpr = pos0.shape
    nbb, d = y_sorted.shape
    seq = nrow * tpr
    assert nrow % _NW == 0
    rows_per_w = nrow // _NW
    mesh = plsc.VectorSubcoreMesh(core_axis_name="c", subcore_axis_name="s")

    @functools.partial(
        pl.kernel, mesh=mesh,
        out_type=jax.ShapeDtypeStruct((seq, d), jnp.float32),
        scratch_types=[
            pltpu.VMEM((tpr,), jnp.int32),
            pltpu.VMEM((tpr,), jnp.int32),
            pltpu.VMEM((tpr, d), jnp.float32),
            pltpu.VMEM((tpr, d), jnp.float32),
            pltpu.SemaphoreType.DMA,
            pltpu.SemaphoreType.DMA,
        ],
        compiler_params=pltpu.CompilerParams(needs_layout_passes=False),
    )
    def k(pos0_hbm, pos1_hbm, ys_hbm, out_hbm,
          i0_v, i1_v, r0_v, r1_v, sem0, sem1):
        wid = lax.axis_index("s") * _NC + lax.axis_index("c")
        for j in range(rows_per_w):
            r = wid * rows_per_w + j
            pltpu.sync_copy(pos0_hbm.at[r], i0_v)
            pltpu.sync_copy(pos1_hbm.at[r], i1_v)
            c0 = pltpu.async_copy(ys_hbm.at[i0_v], r0_v, sem0)
            c1 = pltpu.async_copy(ys_hbm.at[i1_v], r1_v, sem1)
            c0.wait()
            c1.wait()

            def rbody(rr, carry):
                for cc in range(d // _LANES):
                    sl = pl.ds(cc * _LANES, _LANES)
                    r0_v[rr, sl] = r0_v[rr, sl] + r1_v[rr, sl]
                return carry

            lax.fori_loop(0, tpr, rbody, 0)
            pltpu.sync_copy(r0_v, out_hbm.at[pl.ds(r * tpr, tpr)])

    return k(pos0, pos1, y_sorted)


def _ffn_grouped_body(be_ref, nv_ref, first_ref, x_ref, sw_ref, upm_ref,
                      gm_ref, ua_ref, ga_ref, dn_ref, y_ref, dnbf_ref):
    i = pl.program_id(0)
    f32 = jnp.float32
    bf16 = jnp.bfloat16

    @pl.when(first_ref[i] == 1)
    def _():
        dnbf_ref[...] = dn_ref[0].astype(bf16)

    @pl.when(nv_ref[i] > 0)
    def _():
        x = x_ref[...].astype(bf16)
        coef = sw_ref[...]
        tb_up = jax.lax.dot_general(
            x, upm_ref[0], (((1,), (0,)), ((), ())),
            preferred_element_type=f32)
        tb_gate = jax.lax.dot_general(
            x, gm_ref[0], (((1,), (0,)), ((), ())),
            preferred_element_type=f32)
        up = jax.lax.dot_general(
            (coef * tb_up).astype(bf16), ua_ref[0],
            (((1,), (1,)), ((), ())), preferred_element_type=f32)
        gate = jax.lax.dot_general(
            tb_gate.astype(bf16), ga_ref[0],
            (((1,), (1,)), ((), ())), preferred_element_type=f32)
        inter = gate * (0.5 + 0.5 * jnp.tanh(0.5 * gate)) * up
        y_ref[...] = jax.lax.dot_general(
            inter.astype(bf16), dnbf_ref[...],
            (((1,), (1,)), ((), ())), preferred_element_type=f32)


def _ffn_grouped(x_sorted, slot_w, up_mixed, gate_mixed, up_adapters,
                 gate_adapters, down_projections, be, nv, first, nb):
    nbb, d = x_sorted.shape
    p, r = up_adapters.shape[1:]
    grid_spec = pltpu.PrefetchScalarGridSpec(
        num_scalar_prefetch=3,
        grid=(nb,),
        in_specs=[
            pl.BlockSpec((BLK, d), lambda i, be, nv, fs: (i, 0)),
            pl.BlockSpec((BLK, 1), lambda i, be, nv, fs: (i, 0)),
            pl.BlockSpec((1, d, r), lambda i, be, nv, fs: (be[i], 0, 0)),
            pl.BlockSpec((1, d, r), lambda i, be, nv, fs: (be[i], 0, 0)),
            pl.BlockSpec((1, p, r), lambda i, be, nv, fs: (be[i], 0, 0)),
            pl.BlockSpec((1, p, r), lambda i, be, nv, fs: (be[i], 0, 0)),
            pl.BlockSpec((1, d, p), lambda i, be, nv, fs: (be[i], 0, 0)),
        ],
        out_specs=pl.BlockSpec((BLK, d), lambda i, be, nv, fs: (i, 0)),
        scratch_shapes=[
            pltpu.VMEM((d, p), jnp.bfloat16),
        ],
    )
    return pl.pallas_call(
        _ffn_grouped_body,
        grid_spec=grid_spec,
        out_shape=jax.ShapeDtypeStruct((nbb, d), jnp.float32),
    )(be, nv, first, x_sorted, slot_w.reshape(nbb, 1), up_mixed, gate_mixed,
      up_adapters.astype(jnp.bfloat16), gate_adapters.astype(jnp.bfloat16),
      down_projections)


def kernel(hidden_states, expert_indices, expert_weights, up_adapters,
           gate_adapters, up_mixture_logits, gate_mixture_logits,
           down_projections, up_bank, gate_bank):
    seq, d = hidden_states.shape
    g = seq * TOPK
    nb = g // BLK + E
    nbb = nb * BLK

    up_mixed, gate_mixed = _mix_banks(up_mixture_logits, gate_mixture_logits,
                                      up_bank, gate_bank)
    flat_w, slot, be, nv, first = _routing_metadata(
        expert_indices, expert_weights, nb)

    flat_t = jnp.arange(g, dtype=jnp.int32) // TOPK
    x_sorted, slot_w = _dispatch_sc(slot, flat_t, flat_w, hidden_states)

    y_sorted = _ffn_grouped(x_sorted, slot_w, up_mixed, gate_mixed,
                            up_adapters, gate_adapters, down_projections,
                            be, nv, first, nb)

    pos = slot.reshape(seq, TOPK)
    tpr = 32
    pos0 = pos[:, 0].reshape(seq // tpr, tpr)
    pos1 = pos[:, 1].reshape(seq // tpr, tpr)
    return _combine_sc(pos0, pos1, y_sorted)


# speedup vs baseline: 1.3289x; 1.3289x over previous
import functools

import jax
import jax.numpy as jnp
from jax import lax
from jax.experimental import pallas as pl
from jax.experimental.pallas import tpu as pltpu
from jax.experimental.pallas import tpu_sc as plsc

E = 8
TOPK = 2
BLK = 256
_NC = 2
_NS = 16
_NW = _NC * _NS
_LANES = 16


def _mix_body(up_logits_ref, gate_logits_ref, up_bank_ref, gate_bank_ref,
              up_out_ref, gate_out_ref):
    for lref, bref, oref in ((up_logits_ref, up_bank_ref, up_out_ref),
                             (gate_logits_ref, gate_bank_ref, gate_out_ref)):
        logits = lref[...]
        m = jnp.max(logits, axis=1, keepdims=True)
        ex = jnp.exp(logits - m)
        alpha = ex / jnp.sum(ex, axis=1, keepdims=True)
        mixed = jax.lax.dot_general(
            alpha, bref[...], (((1,), (0,)), ((), ())),
            preferred_element_type=jnp.float32)
        oref[...] = mixed.astype(jnp.bfloat16)


def _mix_banks(up_mixture_logits, gate_mixture_logits, up_bank, gate_bank):
    ku, d, r = up_bank.shape
    kg = gate_bank.shape[0]
    up_flat = up_bank.reshape(ku, d * r)
    gate_flat = gate_bank.reshape(kg, d * r)
    up_mixed, gate_mixed = pl.pallas_call(
        _mix_body,
        out_shape=(jax.ShapeDtypeStruct((E, d * r), jnp.bfloat16),
                   jax.ShapeDtypeStruct((E, d * r), jnp.bfloat16)),
    )(up_mixture_logits, gate_mixture_logits, up_flat, gate_flat)
    return up_mixed.reshape(E, d, r), gate_mixed.reshape(E, d, r)


def _routing_metadata(expert_indices, expert_weights, nb):
    i32 = jnp.int32
    g = expert_indices.size
    flat_e = expert_indices.reshape(-1).astype(i32)
    flat_w = expert_weights.reshape(-1)
    oh = flat_e[:, None] == jnp.arange(E, dtype=i32)[None, :]
    ohf = oh.astype(i32)
    ranks_all = jnp.cumsum(ohf, axis=0) - ohf
    rank = jnp.sum(jnp.where(oh, ranks_all, 0), axis=1)
    counts = jnp.sum(ohf, axis=0)
    bpe = (counts + BLK - 1) // BLK
    cumb = jnp.cumsum(bpe)
    base_slot = (jnp.concatenate([jnp.zeros(1, i32), cumb[:-1]]) * BLK)
    slot = base_slot[flat_e] + rank
    used = cumb[-1]
    bids = jnp.arange(nb, dtype=i32)
    raw_be = jnp.searchsorted(cumb, bids, side="right").astype(i32)
    last_e = jnp.searchsorted(cumb, used - 1, side="right").astype(i32)
    be = jnp.minimum(raw_be, last_e)
    block_start = bids * BLK
    nv = jnp.clip(counts[be] - (block_start - base_slot[be]), 0, BLK)
    nv = nv.astype(i32)
    first = ((block_start == base_slot[be]) & (nv > 0)).astype(i32)
    return flat_w, slot, be, nv, first


def _dispatch_sc(slot, flat_t, flat_w, nbb):
    g = slot.shape[0]
    rpw = nbb // _NW
    mesh = plsc.VectorSubcoreMesh(core_axis_name="c", subcore_axis_name="s")

    @functools.partial(
        pl.kernel, mesh=mesh,
        out_type=(jax.ShapeDtypeStruct((nbb,), jnp.int32),
                  jax.ShapeDtypeStruct((nbb,), jnp.float32)),
        scratch_types=[
            pltpu.VMEM((g,), jnp.int32),
            pltpu.VMEM((g,), jnp.int32),
            pltpu.VMEM((g,), jnp.float32),
            pltpu.VMEM((rpw,), jnp.int32),
            pltpu.VMEM((rpw,), jnp.float32),
        ],
        compiler_params=pltpu.CompilerParams(needs_layout_passes=False),
    )
    def k(slot_hbm, tok_hbm, w_hbm, st_hbm, sw_hbm,
          slot_v, tok_v, w_v, st_v, swl_v):
        wid = lax.axis_index("s") * _NC + lax.axis_index("c")
        base = wid * rpw
        pltpu.sync_copy(slot_hbm, slot_v)
        pltpu.sync_copy(tok_hbm, tok_v)
        pltpu.sync_copy(w_hbm, w_v)

        def zbody(i, carry):
            sl = pl.ds(i * _LANES, _LANES)
            st_v[sl] = jnp.zeros((_LANES,), jnp.int32)
            swl_v[sl] = jnp.zeros((_LANES,), jnp.float32)
            return carry

        lax.fori_loop(0, rpw // _LANES, zbody, 0)

        def sbody(i, carry):
            sl = pl.ds(i * _LANES, _LANES)
            rel = slot_v[sl] - base
            msk = (rel >= 0) & (rel < rpw)
            plsc.store_scatter(st_v, [rel], tok_v[sl], mask=msk)
            plsc.store_scatter(swl_v, [rel], w_v[sl], mask=msk)
            return carry

        lax.fori_loop(0, g // _LANES, sbody, 0)
        pltpu.sync_copy(st_v, st_hbm.at[pl.ds(base, rpw)])
        pltpu.sync_copy(swl_v, sw_hbm.at[pl.ds(base, rpw)])

    return k(slot, flat_t, flat_w)


def _combine_sc(pos0, pos1, y_sorted):
    nrow, tpr = pos0.shape
    nbb, d = y_sorted.shape
    seq = nrow * tpr
    assert nrow % _NW == 0
    rows_per_w = nrow // _NW
    mesh = plsc.VectorSubcoreMesh(core_axis_name="c", subcore_axis_name="s")

    @functools.partial(
        pl.kernel, mesh=mesh,
        out_type=jax.ShapeDtypeStruct((seq, d), jnp.float32),
        scratch_types=[
            pltpu.VMEM((tpr,), jnp.int32),
            pltpu.VMEM((tpr,), jnp.int32),
            pltpu.VMEM((tpr, d), jnp.float32),
            pltpu.VMEM((tpr, d), jnp.float32),
            pltpu.SemaphoreType.DMA,
            pltpu.SemaphoreType.DMA,
        ],
        compiler_params=pltpu.CompilerParams(needs_layout_passes=False),
    )
    def k(pos0_hbm, pos1_hbm, ys_hbm, out_hbm,
          i0_v, i1_v, r0_v, r1_v, sem0, sem1):
        wid = lax.axis_index("s") * _NC + lax.axis_index("c")
        for j in range(rows_per_w):
            r = wid * rows_per_w + j
            pltpu.sync_copy(pos0_hbm.at[r], i0_v)
            pltpu.sync_copy(pos1_hbm.at[r], i1_v)
            c0 = pltpu.async_copy(ys_hbm.at[i0_v], r0_v, sem0)
            c1 = pltpu.async_copy(ys_hbm.at[i1_v], r1_v, sem1)
            c0.wait()
            c1.wait()

            def rbody(rr, carry):
                for cc in range(d // _LANES):
                    sl = pl.ds(cc * _LANES, _LANES)
                    r0_v[rr, sl] = r0_v[rr, sl] + r1_v[rr, sl]
                return carry

            lax.fori_loop(0, tpr, rbody, 0)
            pltpu.sync_copy(r0_v, out_hbm.at[pl.ds(r * tpr, tpr)])

    return k(pos0, pos1, y_sorted)


def _ffn_grouped_body(be_ref, nv_ref, first_ref, hid_ref, tok_ref, sw_ref,
                      upm_ref, gm_ref, ua_ref, ga_ref, dn_ref, y_ref,
                      dnbf_ref):
    i = pl.program_id(0)
    f32 = jnp.float32
    bf16 = jnp.bfloat16

    @pl.when(first_ref[i] == 1)
    def _():
        dnbf_ref[...] = dn_ref[0].astype(bf16)

    @pl.when(nv_ref[i] > 0)
    def _():
        seq = hid_ref.shape[0]
        tok = tok_ref[...]
        iota = jax.lax.broadcasted_iota(jnp.int32, (BLK, seq), 1)
        onehot = (tok == iota).astype(bf16)
        x = jax.lax.dot_general(
            onehot, hid_ref[...], (((1,), (0,)), ((), ())),
            preferred_element_type=f32).astype(bf16)
        coef = sw_ref[...]
        tb_up = jax.lax.dot_general(
            x, upm_ref[0], (((1,), (0,)), ((), ())),
            preferred_element_type=f32)
        tb_gate = jax.lax.dot_general(
            x, gm_ref[0], (((1,), (0,)), ((), ())),
            preferred_element_type=f32)
        up = jax.lax.dot_general(
            (coef * tb_up).astype(bf16), ua_ref[0],
            (((1,), (1,)), ((), ())), preferred_element_type=f32)
        gate = jax.lax.dot_general(
            tb_gate.astype(bf16), ga_ref[0],
            (((1,), (1,)), ((), ())), preferred_element_type=f32)
        inter = gate * (0.5 + 0.5 * jnp.tanh(0.5 * gate)) * up
        y_ref[...] = jax.lax.dot_general(
            inter.astype(bf16), dnbf_ref[...],
            (((1,), (1,)), ((), ())), preferred_element_type=f32)


def _ffn_grouped(hidden_bf, slot_token, slot_w, up_mixed, gate_mixed,
                 up_adapters, gate_adapters, down_projections, be, nv, first,
                 nb):
    seq, d = hidden_bf.shape
    nbb = slot_token.shape[0]
    p, r = up_adapters.shape[1:]
    grid_spec = pltpu.PrefetchScalarGridSpec(
        num_scalar_prefetch=3,
        grid=(nb,),
        in_specs=[
            pl.BlockSpec((seq, d), lambda i, be, nv, fs: (0, 0)),
            pl.BlockSpec((BLK, 1), lambda i, be, nv, fs: (i, 0)),
            pl.BlockSpec((BLK, 1), lambda i, be, nv, fs: (i, 0)),
            pl.BlockSpec((1, d, r), lambda i, be, nv, fs: (be[i], 0, 0)),
            pl.BlockSpec((1, d, r), lambda i, be, nv, fs: (be[i], 0, 0)),
            pl.BlockSpec((1, p, r), lambda i, be, nv, fs: (be[i], 0, 0)),
            pl.BlockSpec((1, p, r), lambda i, be, nv, fs: (be[i], 0, 0)),
            pl.BlockSpec((1, d, p), lambda i, be, nv, fs: (be[i], 0, 0)),
        ],
        out_specs=pl.BlockSpec((BLK, d), lambda i, be, nv, fs: (i, 0)),
        scratch_shapes=[
            pltpu.VMEM((d, p), jnp.bfloat16),
        ],
    )
    return pl.pallas_call(
        _ffn_grouped_body,
        grid_spec=grid_spec,
        out_shape=jax.ShapeDtypeStruct((nbb, d), jnp.float32),
        compiler_params=pltpu.CompilerParams(
            vmem_limit_bytes=64 * 1024 * 1024),
    )(be, nv, first, hidden_bf, slot_token.reshape(nbb, 1),
      slot_w.reshape(nbb, 1), up_mixed, gate_mixed,
      up_adapters.astype(jnp.bfloat16), gate_adapters.astype(jnp.bfloat16),
      down_projections)


def kernel(hidden_states, expert_indices, expert_weights, up_adapters,
           gate_adapters, up_mixture_logits, gate_mixture_logits,
           down_projections, up_bank, gate_bank):
    seq, d = hidden_states.shape
    g = seq * TOPK
    nb = g // BLK + E
    nbb = nb * BLK

    up_mixed, gate_mixed = _mix_banks(up_mixture_logits, gate_mixture_logits,
                                      up_bank, gate_bank)
    flat_w, slot, be, nv, first = _routing_metadata(
        expert_indices, expert_weights, nb)

    flat_t = jnp.arange(g, dtype=jnp.int32) // TOPK
    slot_token, slot_w = _dispatch_sc(slot, flat_t, flat_w, nbb)

    y_sorted = _ffn_grouped(hidden_states.astype(jnp.bfloat16), slot_token,
                            slot_w, up_mixed, gate_mixed, up_adapters,
                            gate_adapters, down_projections, be, nv, first,
                            nb)

    pos = slot.reshape(seq, TOPK)
    tpr = 32
    pos0 = pos[:, 0].reshape(seq // tpr, tpr)
    pos1 = pos[:, 1].reshape(seq // tpr, tpr)
    return _combine_sc(pos0, pos1, y_sorted)


# speedup vs baseline: 1.3535x; 1.0185x over previous
import functools

import jax
import jax.numpy as jnp
from jax import lax
from jax.experimental import pallas as pl
from jax.experimental.pallas import tpu as pltpu
from jax.experimental.pallas import tpu_sc as plsc

E = 8
TOPK = 2
BLK = 256
_NC = 2
_NS = 16
_NW = _NC * _NS
_LANES = 16


def _mix_body(up_logits_ref, gate_logits_ref, up_bank_ref, gate_bank_ref,
              up_out_ref, gate_out_ref):
    for lref, bref, oref in ((up_logits_ref, up_bank_ref, up_out_ref),
                             (gate_logits_ref, gate_bank_ref, gate_out_ref)):
        logits = lref[...]
        m = jnp.max(logits, axis=1, keepdims=True)
        ex = jnp.exp(logits - m)
        alpha = ex / jnp.sum(ex, axis=1, keepdims=True)
        mixed = jax.lax.dot_general(
            alpha, bref[...], (((1,), (0,)), ((), ())),
            preferred_element_type=jnp.float32)
        oref[...] = mixed.astype(jnp.bfloat16)


def _mix_banks(up_mixture_logits, gate_mixture_logits, up_bank, gate_bank):
    ku, d, r = up_bank.shape
    kg = gate_bank.shape[0]
    up_flat = up_bank.reshape(ku, d * r)
    gate_flat = gate_bank.reshape(kg, d * r)
    up_mixed, gate_mixed = pl.pallas_call(
        _mix_body,
        out_shape=(jax.ShapeDtypeStruct((E, d * r), jnp.bfloat16),
                   jax.ShapeDtypeStruct((E, d * r), jnp.bfloat16)),
    )(up_mixture_logits, gate_mixture_logits, up_flat, gate_flat)
    return up_mixed.reshape(E, d, r), gate_mixed.reshape(E, d, r)


def _routing_metadata(expert_indices, expert_weights, nb):
    i32 = jnp.int32
    g = expert_indices.size
    flat_e = expert_indices.reshape(-1).astype(i32)
    flat_w = expert_weights.reshape(-1)
    oh = flat_e[:, None] == jnp.arange(E, dtype=i32)[None, :]
    ohf = oh.astype(i32)
    ranks_all = jnp.cumsum(ohf, axis=0) - ohf
    rank = jnp.sum(jnp.where(oh, ranks_all, 0), axis=1)
    counts = jnp.sum(ohf, axis=0)
    bpe = (counts + BLK - 1) // BLK
    cumb = jnp.cumsum(bpe)
    base_slot = (jnp.concatenate([jnp.zeros(1, i32), cumb[:-1]]) * BLK)
    slot = base_slot[flat_e] + rank
    used = cumb[-1]
    bids = jnp.arange(nb, dtype=i32)
    raw_be = jnp.searchsorted(cumb, bids, side="right").astype(i32)
    last_e = jnp.searchsorted(cumb, used - 1, side="right").astype(i32)
    be = jnp.minimum(raw_be, last_e)
    block_start = bids * BLK
    nv = jnp.clip(counts[be] - (block_start - base_slot[be]), 0, BLK)
    nv = nv.astype(i32)
    first = ((block_start == base_slot[be]) & (nv > 0)).astype(i32)
    return flat_w, slot, be, nv, first


def _dispatch_sc(slot, flat_t, flat_w, nbb):
    g = slot.shape[0]
    rpw = nbb // _NW
    mesh = plsc.VectorSubcoreMesh(core_axis_name="c", subcore_axis_name="s")

    @functools.partial(
        pl.kernel, mesh=mesh,
        out_type=(jax.ShapeDtypeStruct((nbb,), jnp.int32),
                  jax.ShapeDtypeStruct((nbb,), jnp.float32)),
        scratch_types=[
            pltpu.VMEM((g,), jnp.int32),
            pltpu.VMEM((g,), jnp.int32),
            pltpu.VMEM((g,), jnp.float32),
            pltpu.VMEM((rpw,), jnp.int32),
            pltpu.VMEM((rpw,), jnp.float32),
        ],
        compiler_params=pltpu.CompilerParams(needs_layout_passes=False),
    )
    def k(slot_hbm, tok_hbm, w_hbm, st_hbm, sw_hbm,
          slot_v, tok_v, w_v, st_v, swl_v):
        wid = lax.axis_index("s") * _NC + lax.axis_index("c")
        base = wid * rpw
        pltpu.sync_copy(slot_hbm, slot_v)
        pltpu.sync_copy(tok_hbm, tok_v)
        pltpu.sync_copy(w_hbm, w_v)

        def zbody(i, carry):
            sl = pl.ds(i * _LANES, _LANES)
            st_v[sl] = jnp.zeros((_LANES,), jnp.int32)
            swl_v[sl] = jnp.zeros((_LANES,), jnp.float32)
            return carry

        lax.fori_loop(0, rpw // _LANES, zbody, 0)

        def sbody(i, carry):
            sl = pl.ds(i * _LANES, _LANES)
            rel = slot_v[sl] - base
            msk = (rel >= 0) & (rel < rpw)
            plsc.store_scatter(st_v, [rel], tok_v[sl], mask=msk)
            plsc.store_scatter(swl_v, [rel], w_v[sl], mask=msk)
            return carry

        lax.fori_loop(0, g // _LANES, sbody, 0)
        pltpu.sync_copy(st_v, st_hbm.at[pl.ds(base, rpw)])
        pltpu.sync_copy(swl_v, sw_hbm.at[pl.ds(base, rpw)])

    return k(slot, flat_t, flat_w)


def _combine_sc(pos0, pos1, y_sorted):
    nrow, tpr = pos0.shape
    nbb, d = y_sorted.shape
    seq = nrow * tpr
    assert nrow % _NW == 0
    rows_per_w = nrow // _NW
    mesh = plsc.VectorSubcoreMesh(core_axis_name="c", subcore_axis_name="s")

    @functools.partial(
        pl.kernel, mesh=mesh,
        out_type=jax.ShapeDtypeStruct((seq, d), jnp.float32),
        scratch_types=[
            pltpu.VMEM((tpr,), jnp.int32),
            pltpu.VMEM((tpr,), jnp.int32),
            pltpu.VMEM((tpr, d), jnp.float32),
            pltpu.VMEM((tpr, d), jnp.float32),
            pltpu.SemaphoreType.DMA,
            pltpu.SemaphoreType.DMA,
        ],
        compiler_params=pltpu.CompilerParams(needs_layout_passes=False),
    )
    def k(pos0_hbm, pos1_hbm, ys_hbm, out_hbm,
          i0_v, i1_v, r0_v, r1_v, sem0, sem1):
        wid = lax.axis_index("s") * _NC + lax.axis_index("c")
        for j in range(rows_per_w):
            r = wid * rows_per_w + j
            pltpu.sync_copy(pos0_hbm.at[r], i0_v)
            pltpu.sync_copy(pos1_hbm.at[r], i1_v)
            c0 = pltpu.async_copy(ys_hbm.at[i0_v], r0_v, sem0)
            c1 = pltpu.async_copy(ys_hbm.at[i1_v], r1_v, sem1)
            c0.wait()
            c1.wait()

            def rbody(rr, carry):
                for cc in range(d // _LANES):
                    sl = pl.ds(cc * _LANES, _LANES)
                    r0_v[rr, sl] = r0_v[rr, sl] + r1_v[rr, sl]
                return carry

            lax.fori_loop(0, tpr, rbody, 0)
            pltpu.sync_copy(r0_v, out_hbm.at[pl.ds(r * tpr, tpr)])

    return k(pos0, pos1, y_sorted)


def _ffn_grouped_body(be_ref, nv_ref, first_ref, hid_ref, tok_ref, sw_ref,
                      upm_ref, gm_ref, ua_ref, ga_ref, dn_ref, y_ref,
                      uabf_ref, gabf_ref, dnbf_ref):
    i = pl.program_id(0)
    f32 = jnp.float32
    bf16 = jnp.bfloat16

    @pl.when(first_ref[i] == 1)
    def _():
        uabf_ref[...] = ua_ref[0].astype(bf16)
        gabf_ref[...] = ga_ref[0].astype(bf16)
        dnbf_ref[...] = dn_ref[0].astype(bf16)

    @pl.when(nv_ref[i] > 0)
    def _():
        seq = hid_ref.shape[0]
        tok = tok_ref[...]
        iota = jax.lax.broadcasted_iota(jnp.int32, (BLK, seq), 1)
        onehot = (tok == iota).astype(bf16)
        x = jax.lax.dot_general(
            onehot, hid_ref[...], (((1,), (0,)), ((), ())),
            preferred_element_type=f32).astype(bf16)
        coef = sw_ref[...]
        tb_up = jax.lax.dot_general(
            x, upm_ref[0], (((1,), (0,)), ((), ())),
            preferred_element_type=f32)
        tb_gate = jax.lax.dot_general(
            x, gm_ref[0], (((1,), (0,)), ((), ())),
            preferred_element_type=f32)
        up = jax.lax.dot_general(
            (coef * tb_up).astype(bf16), uabf_ref[...],
            (((1,), (1,)), ((), ())), preferred_element_type=f32)
        gate = jax.lax.dot_general(
            tb_gate.astype(bf16), gabf_ref[...],
            (((1,), (1,)), ((), ())), preferred_element_type=f32)
        inter = gate * (0.5 + 0.5 * jnp.tanh(0.5 * gate)) * up
        y_ref[...] = jax.lax.dot_general(
            inter.astype(bf16), dnbf_ref[...],
            (((1,), (1,)), ((), ())), preferred_element_type=f32)


def _ffn_grouped(hidden_bf, slot_token, slot_w, up_mixed, gate_mixed,
                 up_adapters, gate_adapters, down_projections, be, nv, first,
                 nb):
    seq, d = hidden_bf.shape
    nbb = slot_token.shape[0]
    p, r = up_adapters.shape[1:]
    grid_spec = pltpu.PrefetchScalarGridSpec(
        num_scalar_prefetch=3,
        grid=(nb,),
        in_specs=[
            pl.BlockSpec((seq, d), lambda i, be, nv, fs: (0, 0)),
            pl.BlockSpec((BLK, 1), lambda i, be, nv, fs: (i, 0)),
            pl.BlockSpec((BLK, 1), lambda i, be, nv, fs: (i, 0)),
            pl.BlockSpec((1, d, r), lambda i, be, nv, fs: (be[i], 0, 0)),
            pl.BlockSpec((1, d, r), lambda i, be, nv, fs: (be[i], 0, 0)),
            pl.BlockSpec((1, p, r), lambda i, be, nv, fs: (be[i], 0, 0)),
            pl.BlockSpec((1, p, r), lambda i, be, nv, fs: (be[i], 0, 0)),
            pl.BlockSpec((1, d, p), lambda i, be, nv, fs: (be[i], 0, 0)),
        ],
        out_specs=pl.BlockSpec((BLK, d), lambda i, be, nv, fs: (i, 0)),
        scratch_shapes=[
            pltpu.VMEM((p, r), jnp.bfloat16),
            pltpu.VMEM((p, r), jnp.bfloat16),
            pltpu.VMEM((d, p), jnp.bfloat16),
        ],
    )
    return pl.pallas_call(
        _ffn_grouped_body,
        grid_spec=grid_spec,
        out_shape=jax.ShapeDtypeStruct((nbb, d), jnp.float32),
        compiler_params=pltpu.CompilerParams(
            vmem_limit_bytes=64 * 1024 * 1024),
    )(be, nv, first, hidden_bf, slot_token.reshape(nbb, 1),
      slot_w.reshape(nbb, 1), up_mixed, gate_mixed,
      up_adapters, gate_adapters, down_projections)


def kernel(hidden_states, expert_indices, expert_weights, up_adapters,
           gate_adapters, up_mixture_logits, gate_mixture_logits,
           down_projections, up_bank, gate_bank):
    seq, d = hidden_states.shape
    g = seq * TOPK
    nb = g // BLK + E
    nbb = nb * BLK

    up_mixed, gate_mixed = _mix_banks(up_mixture_logits, gate_mixture_logits,
                                      up_bank, gate_bank)
    flat_w, slot, be, nv, first = _routing_metadata(
        expert_indices, expert_weights, nb)

    flat_t = jnp.arange(g, dtype=jnp.int32) // TOPK
    slot_token, slot_w = _dispatch_sc(slot, flat_t, flat_w, nbb)

    y_sorted = _ffn_grouped(hidden_states.astype(jnp.bfloat16), slot_token,
                            slot_w, up_mixed, gate_mixed, up_adapters,
                            gate_adapters, down_projections, be, nv, first,
                            nb)

    pos = slot.reshape(seq, TOPK)
    tpr = 32
    pos0 = pos[:, 0].reshape(seq // tpr, tpr)
    pos1 = pos[:, 1].reshape(seq // tpr, tpr)
    return _combine_sc(pos0, pos1, y_sorted)


# speedup vs baseline: 1.3798x; 1.0194x over previous
import functools

import jax
import jax.numpy as jnp
from jax import lax
from jax.experimental import pallas as pl
from jax.experimental.pallas import tpu as pltpu
from jax.experimental.pallas import tpu_sc as plsc

E = 8
TOPK = 2
BLK = 256
_NC = 2
_NS = 16
_NW = _NC * _NS
_LANES = 16


def _mix_body(up_logits_ref, gate_logits_ref, up_bank_ref, gate_bank_ref,
              up_out_ref, gate_out_ref):
    for lref, bref, oref in ((up_logits_ref, up_bank_ref, up_out_ref),
                             (gate_logits_ref, gate_bank_ref, gate_out_ref)):
        logits = lref[...]
        m = jnp.max(logits, axis=1, keepdims=True)
        ex = jnp.exp(logits - m)
        alpha = ex / jnp.sum(ex, axis=1, keepdims=True)
        mixed = jax.lax.dot_general(
            alpha, bref[...], (((1,), (0,)), ((), ())),
            preferred_element_type=jnp.float32)
        oref[...] = mixed.astype(jnp.bfloat16)


def _mix_banks(up_mixture_logits, gate_mixture_logits, up_bank, gate_bank):
    ku, d, r = up_bank.shape
    kg = gate_bank.shape[0]
    up_flat = up_bank.reshape(ku, d * r)
    gate_flat = gate_bank.reshape(kg, d * r)
    up_mixed, gate_mixed = pl.pallas_call(
        _mix_body,
        out_shape=(jax.ShapeDtypeStruct((E, d * r), jnp.bfloat16),
                   jax.ShapeDtypeStruct((E, d * r), jnp.bfloat16)),
    )(up_mixture_logits, gate_mixture_logits, up_flat, gate_flat)
    return up_mixed.reshape(E, d, r), gate_mixed.reshape(E, d, r)


def _routing_metadata(expert_indices, expert_weights, nb):
    i32 = jnp.int32
    g = expert_indices.size
    flat_e = expert_indices.reshape(-1).astype(i32)
    flat_w = expert_weights.reshape(-1)
    oh = flat_e[:, None] == jnp.arange(E, dtype=i32)[None, :]
    ohf = oh.astype(i32)
    ranks_all = jnp.cumsum(ohf, axis=0) - ohf
    rank = jnp.sum(jnp.where(oh, ranks_all, 0), axis=1)
    counts = jnp.sum(ohf, axis=0)
    bpe = (counts + BLK - 1) // BLK
    cumb = jnp.cumsum(bpe)
    base_slot = (jnp.concatenate([jnp.zeros(1, i32), cumb[:-1]]) * BLK)
    slot = base_slot[flat_e] + rank
    used = cumb[-1]
    bids = jnp.arange(nb, dtype=i32)
    raw_be = jnp.searchsorted(cumb, bids, side="right").astype(i32)
    last_e = jnp.searchsorted(cumb, used - 1, side="right").astype(i32)
    be = jnp.minimum(raw_be, last_e)
    block_start = bids * BLK
    nv = jnp.clip(counts[be] - (block_start - base_slot[be]), 0, BLK)
    nv = nv.astype(i32)
    first = ((block_start == base_slot[be]) & (nv > 0)).astype(i32)
    return flat_w, slot, be, nv, first


def _dispatch_sc(slot, flat_t, flat_w, nbb):
    g = slot.shape[0]
    rpw = nbb // _NW
    mesh = plsc.VectorSubcoreMesh(core_axis_name="c", subcore_axis_name="s")

    @functools.partial(
        pl.kernel, mesh=mesh,
        out_type=(jax.ShapeDtypeStruct((nbb,), jnp.int32),
                  jax.ShapeDtypeStruct((nbb,), jnp.float32)),
        scratch_types=[
            pltpu.VMEM((g,), jnp.int32),
            pltpu.VMEM((g,), jnp.int32),
            pltpu.VMEM((g,), jnp.float32),
            pltpu.VMEM((rpw,), jnp.int32),
            pltpu.VMEM((rpw,), jnp.float32),
        ],
        compiler_params=pltpu.CompilerParams(needs_layout_passes=False),
    )
    def k(slot_hbm, tok_hbm, w_hbm, st_hbm, sw_hbm,
          slot_v, tok_v, w_v, st_v, swl_v):
        wid = lax.axis_index("s") * _NC + lax.axis_index("c")
        base = wid * rpw
        pltpu.sync_copy(slot_hbm, slot_v)
        pltpu.sync_copy(tok_hbm, tok_v)
        pltpu.sync_copy(w_hbm, w_v)

        def zbody(i, carry):
            sl = pl.ds(i * _LANES, _LANES)
            st_v[sl] = jnp.zeros((_LANES,), jnp.int32)
            swl_v[sl] = jnp.zeros((_LANES,), jnp.float32)
            return carry

        lax.fori_loop(0, rpw // _LANES, zbody, 0)

        def sbody(i, carry):
            sl = pl.ds(i * _LANES, _LANES)
            rel = slot_v[sl] - base
            msk = (rel >= 0) & (rel < rpw)
            plsc.store_scatter(st_v, [rel], tok_v[sl], mask=msk)
            plsc.store_scatter(swl_v, [rel], w_v[sl], mask=msk)
            return carry

        lax.fori_loop(0, g // _LANES, sbody, 0)
        pltpu.sync_copy(st_v, st_hbm.at[pl.ds(base, rpw)])
        pltpu.sync_copy(swl_v, sw_hbm.at[pl.ds(base, rpw)])

    return k(slot, flat_t, flat_w)


def _combine_sc(pos0, pos1, y_sorted):
    nrow, tpr = pos0.shape
    nbb, d = y_sorted.shape
    seq = nrow * tpr
    assert nrow % _NW == 0
    rows_per_w = nrow // _NW
    mesh = plsc.VectorSubcoreMesh(core_axis_name="c", subcore_axis_name="s")

    @functools.partial(
        pl.kernel, mesh=mesh,
        out_type=jax.ShapeDtypeStruct((seq, d), jnp.float32),
        scratch_types=[
            pltpu.VMEM((tpr,), jnp.int32),
            pltpu.VMEM((tpr,), jnp.int32),
            pltpu.VMEM((tpr, d), jnp.float32),
            pltpu.VMEM((tpr, d), jnp.float32),
            pltpu.SemaphoreType.DMA,
            pltpu.SemaphoreType.DMA,
        ],
        compiler_params=pltpu.CompilerParams(needs_layout_passes=False),
    )
    def k(pos0_hbm, pos1_hbm, ys_hbm, out_hbm,
          i0_v, i1_v, r0_v, r1_v, sem0, sem1):
        wid = lax.axis_index("s") * _NC + lax.axis_index("c")
        for j in range(rows_per_w):
            r = wid * rows_per_w + j
            pltpu.sync_copy(pos0_hbm.at[r], i0_v)
            pltpu.sync_copy(pos1_hbm.at[r], i1_v)
            c0 = pltpu.async_copy(ys_hbm.at[i0_v], r0_v, sem0)
            c1 = pltpu.async_copy(ys_hbm.at[i1_v], r1_v, sem1)
            c0.wait()
            c1.wait()

            def rbody(rr, carry):
                for cc in range(d // _LANES):
                    sl = pl.ds(cc * _LANES, _LANES)
                    r0_v[rr, sl] = r0_v[rr, sl] + r1_v[rr, sl]
                return carry

            lax.fori_loop(0, tpr, rbody, 0)
            pltpu.sync_copy(r0_v, out_hbm.at[pl.ds(r * tpr, tpr)])

    return k(pos0, pos1, y_sorted)


def _ffn_grouped_body(be_ref, nv_ref, first_ref, hid_ref, tok_ref, sw_ref,
                      upm_ref, gm_ref, ua_ref, ga_ref, dn_ref, y_ref,
                      uabf_ref, gabf_ref, dnbf_ref):
    i = pl.program_id(0)
    f32 = jnp.float32
    bf16 = jnp.bfloat16

    @pl.when(first_ref[i] == 1)
    def _():
        uabf_ref[...] = ua_ref[0].astype(bf16)
        gabf_ref[...] = ga_ref[0].astype(bf16)
        dnbf_ref[...] = dn_ref[0].astype(bf16)

    @pl.when(nv_ref[i] > 0)
    def _():
        seq = hid_ref.shape[0]
        tok = tok_ref[...]
        iota = jax.lax.broadcasted_iota(jnp.int32, (BLK, seq), 1)
        onehot = (tok == iota).astype(bf16)
        x = jax.lax.dot_general(
            onehot, hid_ref[...], (((1,), (0,)), ((), ())),
            preferred_element_type=f32).astype(bf16)
        coef = sw_ref[...]
        tb_up = jax.lax.dot_general(
            x, upm_ref[0], (((1,), (0,)), ((), ())),
            preferred_element_type=f32)
        tb_gate = jax.lax.dot_general(
            x, gm_ref[0], (((1,), (0,)), ((), ())),
            preferred_element_type=f32)
        up = jax.lax.dot_general(
            (coef * tb_up).astype(bf16), uabf_ref[...],
            (((1,), (1,)), ((), ())), preferred_element_type=f32)
        gate = jax.lax.dot_general(
            tb_gate.astype(bf16), gabf_ref[...],
            (((1,), (1,)), ((), ())), preferred_element_type=f32)
        gate16 = gate.astype(bf16)
        up16 = up.astype(bf16)
        t = jnp.tanh(gate16 * jnp.bfloat16(0.5))
        inter = gate16 * (jnp.bfloat16(0.5) + jnp.bfloat16(0.5) * t) * up16
        y_ref[...] = jax.lax.dot_general(
            inter, dnbf_ref[...],
            (((1,), (1,)), ((), ())), preferred_element_type=f32)


def _ffn_grouped(hidden_bf, slot_token, slot_w, up_mixed, gate_mixed,
                 up_adapters, gate_adapters, down_projections, be, nv, first,
                 nb):
    seq, d = hidden_bf.shape
    nbb = slot_token.shape[0]
    p, r = up_adapters.shape[1:]
    grid_spec = pltpu.PrefetchScalarGridSpec(
        num_scalar_prefetch=3,
        grid=(nb,),
        in_specs=[
            pl.BlockSpec((seq, d), lambda i, be, nv, fs: (0, 0)),
            pl.BlockSpec((BLK, 1), lambda i, be, nv, fs: (i, 0)),
            pl.BlockSpec((BLK, 1), lambda i, be, nv, fs: (i, 0)),
            pl.BlockSpec((1, d, r), lambda i, be, nv, fs: (be[i], 0, 0)),
            pl.BlockSpec((1, d, r), lambda i, be, nv, fs: (be[i], 0, 0)),
            pl.BlockSpec((1, p, r), lambda i, be, nv, fs: (be[i], 0, 0)),
            pl.BlockSpec((1, p, r), lambda i, be, nv, fs: (be[i], 0, 0)),
            pl.BlockSpec((1, d, p), lambda i, be, nv, fs: (be[i], 0, 0)),
        ],
        out_specs=pl.BlockSpec((BLK, d), lambda i, be, nv, fs: (i, 0)),
        scratch_shapes=[
            pltpu.VMEM((p, r), jnp.bfloat16),
            pltpu.VMEM((p, r), jnp.bfloat16),
            pltpu.VMEM((d, p), jnp.bfloat16),
        ],
    )
    return pl.pallas_call(
        _ffn_grouped_body,
        grid_spec=grid_spec,
        out_shape=jax.ShapeDtypeStruct((nbb, d), jnp.float32),
        compiler_params=pltpu.CompilerParams(
            vmem_limit_bytes=64 * 1024 * 1024),
    )(be, nv, first, hidden_bf, slot_token.reshape(nbb, 1),
      slot_w.reshape(nbb, 1), up_mixed, gate_mixed,
      up_adapters, gate_adapters, down_projections)


def kernel(hidden_states, expert_indices, expert_weights, up_adapters,
           gate_adapters, up_mixture_logits, gate_mixture_logits,
           down_projections, up_bank, gate_bank):
    seq, d = hidden_states.shape
    g = seq * TOPK
    nb = g // BLK + E
    nbb = nb * BLK

    up_mixed, gate_mixed = _mix_banks(up_mixture_logits, gate_mixture_logits,
                                      up_bank, gate_bank)
    flat_w, slot, be, nv, first = _routing_metadata(
        expert_indices, expert_weights, nb)

    flat_t = jnp.arange(g, dtype=jnp.int32) // TOPK
    slot_token, slot_w = _dispatch_sc(slot, flat_t, flat_w, nbb)

    y_sorted = _ffn_grouped(hidden_states.astype(jnp.bfloat16), slot_token,
                            slot_w, up_mixed, gate_mixed, up_adapters,
                            gate_adapters, down_projections, be, nv, first,
                            nb)

    pos = slot.reshape(seq, TOPK)
    tpr = 32
    pos0 = pos[:, 0].reshape(seq // tpr, tpr)
    pos1 = pos[:, 1].reshape(seq // tpr, tpr)
    return _combine_sc(pos0, pos1, y_sorted)


# speedup vs baseline: 1.3841x; 1.0031x over previous
import functools

import jax
import jax.numpy as jnp
from jax import lax
from jax.experimental import pallas as pl
from jax.experimental.pallas import tpu as pltpu
from jax.experimental.pallas import tpu_sc as plsc

E = 8
TOPK = 2
BLK = 256
_NC = 2
_NS = 16
_NW = _NC * _NS
_LANES = 16


def _mix_body(up_logits_ref, gate_logits_ref, up_bank_ref, gate_bank_ref,
              up_out_ref, gate_out_ref):
    for lref, bref, oref in ((up_logits_ref, up_bank_ref, up_out_ref),
                             (gate_logits_ref, gate_bank_ref, gate_out_ref)):
        logits = lref[...]
        m = jnp.max(logits, axis=1, keepdims=True)
        ex = jnp.exp(logits - m)
        alpha = ex / jnp.sum(ex, axis=1, keepdims=True)
        mixed = jax.lax.dot_general(
            alpha, bref[...], (((1,), (0,)), ((), ())),
            preferred_element_type=jnp.float32)
        oref[...] = mixed.astype(jnp.bfloat16)


def _mix_banks(up_mixture_logits, gate_mixture_logits, up_bank, gate_bank):
    ku, d, r = up_bank.shape
    kg = gate_bank.shape[0]
    up_flat = up_bank.reshape(ku, d * r)
    gate_flat = gate_bank.reshape(kg, d * r)
    up_mixed, gate_mixed = pl.pallas_call(
        _mix_body,
        out_shape=(jax.ShapeDtypeStruct((E, d * r), jnp.bfloat16),
                   jax.ShapeDtypeStruct((E, d * r), jnp.bfloat16)),
    )(up_mixture_logits, gate_mixture_logits, up_flat, gate_flat)
    return up_mixed.reshape(E, d, r), gate_mixed.reshape(E, d, r)


def _routing_metadata(expert_indices, expert_weights, nb):
    i32 = jnp.int32
    g = expert_indices.size
    flat_e = expert_indices.reshape(-1).astype(i32)
    flat_w = expert_weights.reshape(-1)
    oh = flat_e[:, None] == jnp.arange(E, dtype=i32)[None, :]
    ohf = oh.astype(i32)
    ranks_all = jnp.cumsum(ohf, axis=0) - ohf
    rank = jnp.sum(jnp.where(oh, ranks_all, 0), axis=1)
    counts = jnp.sum(ohf, axis=0)
    bpe = (counts + BLK - 1) // BLK
    cumb = jnp.cumsum(bpe)
    base_slot = (jnp.concatenate([jnp.zeros(1, i32), cumb[:-1]]) * BLK)
    slot = base_slot[flat_e] + rank
    used = cumb[-1]
    bids = jnp.arange(nb, dtype=i32)
    raw_be = jnp.searchsorted(cumb, bids, side="right").astype(i32)
    last_e = jnp.searchsorted(cumb, used - 1, side="right").astype(i32)
    be = jnp.minimum(raw_be, last_e)
    block_start = bids * BLK
    nv = jnp.clip(counts[be] - (block_start - base_slot[be]), 0, BLK)
    nv = nv.astype(i32)
    first = ((block_start == base_slot[be]) & (nv > 0)).astype(i32)
    return flat_w, slot, be, nv, first


def _dispatch_sc(slot, flat_t, flat_w, nbb):
    g = slot.shape[0]
    rpw = nbb // _NW
    mesh = plsc.VectorSubcoreMesh(core_axis_name="c", subcore_axis_name="s")

    @functools.partial(
        pl.kernel, mesh=mesh,
        out_type=(jax.ShapeDtypeStruct((nbb,), jnp.int32),
                  jax.ShapeDtypeStruct((nbb,), jnp.float32)),
        scratch_types=[
            pltpu.VMEM((g,), jnp.int32),
            pltpu.VMEM((g,), jnp.int32),
            pltpu.VMEM((g,), jnp.float32),
            pltpu.VMEM((rpw,), jnp.int32),
            pltpu.VMEM((rpw,), jnp.float32),
        ],
        compiler_params=pltpu.CompilerParams(needs_layout_passes=False),
    )
    def k(slot_hbm, tok_hbm, w_hbm, st_hbm, sw_hbm,
          slot_v, tok_v, w_v, st_v, swl_v):
        wid = lax.axis_index("s") * _NC + lax.axis_index("c")
        base = wid * rpw
        pltpu.sync_copy(slot_hbm, slot_v)
        pltpu.sync_copy(tok_hbm, tok_v)
        pltpu.sync_copy(w_hbm, w_v)

        def zbody(i, carry):
            sl = pl.ds(i * _LANES, _LANES)
            st_v[sl] = jnp.zeros((_LANES,), jnp.int32)
            swl_v[sl] = jnp.zeros((_LANES,), jnp.float32)
            return carry

        lax.fori_loop(0, rpw // _LANES, zbody, 0)

        def sbody(i, carry):
            sl = pl.ds(i * _LANES, _LANES)
            rel = slot_v[sl] - base
            msk = (rel >= 0) & (rel < rpw)
            plsc.store_scatter(st_v, [rel], tok_v[sl], mask=msk)
            plsc.store_scatter(swl_v, [rel], w_v[sl], mask=msk)
            return carry

        lax.fori_loop(0, g // _LANES, sbody, 0)
        pltpu.sync_copy(st_v, st_hbm.at[pl.ds(base, rpw)])
        pltpu.sync_copy(swl_v, sw_hbm.at[pl.ds(base, rpw)])

    return k(slot, flat_t, flat_w)


def _combine_sc(pos0, pos1, y_sorted):
    nrow, tpr = pos0.shape
    nbb, d = y_sorted.shape
    seq = nrow * tpr
    assert nrow % _NW == 0
    rows_per_w = nrow // _NW
    mesh = plsc.VectorSubcoreMesh(core_axis_name="c", subcore_axis_name="s")

    @functools.partial(
        pl.kernel, mesh=mesh,
        out_type=jax.ShapeDtypeStruct((seq, d), jnp.float32),
        scratch_types=[
            pltpu.VMEM((tpr,), jnp.int32),
            pltpu.VMEM((tpr,), jnp.int32),
            pltpu.VMEM((tpr, d), jnp.float32),
            pltpu.VMEM((tpr, d), jnp.float32),
            pltpu.SemaphoreType.DMA,
            pltpu.SemaphoreType.DMA,
        ],
        compiler_params=pltpu.CompilerParams(needs_layout_passes=False),
    )
    def k(pos0_hbm, pos1_hbm, ys_hbm, out_hbm,
          i0_v, i1_v, r0_v, r1_v, sem0, sem1):
        wid = lax.axis_index("s") * _NC + lax.axis_index("c")
        for j in range(rows_per_w):
            r = wid * rows_per_w + j
            pltpu.sync_copy(pos0_hbm.at[r], i0_v)
            pltpu.sync_copy(pos1_hbm.at[r], i1_v)
            c0 = pltpu.async_copy(ys_hbm.at[i0_v], r0_v, sem0)
            c1 = pltpu.async_copy(ys_hbm.at[i1_v], r1_v, sem1)
            c0.wait()
            c1.wait()

            def rbody(rr, carry):
                for cc in range(d // _LANES):
                    sl = pl.ds(cc * _LANES, _LANES)
                    r0_v[rr, sl] = r0_v[rr, sl] + r1_v[rr, sl]
                return carry

            lax.fori_loop(0, tpr, rbody, 0)
            pltpu.sync_copy(r0_v, out_hbm.at[pl.ds(r * tpr, tpr)])

    return k(pos0, pos1, y_sorted)


def _ffn_grouped_body(be_ref, nv_ref, first_ref, hid_ref, tok_ref, sw_ref,
                      upm_ref, gm_ref, ua_ref, ga_ref, dn_ref, y_ref,
                      uabf_ref, gabf_ref, dnbf_ref):
    i = pl.program_id(0)
    f32 = jnp.float32
    bf16 = jnp.bfloat16

    @pl.when(first_ref[i] == 1)
    def _():
        uabf_ref[...] = ua_ref[0].astype(bf16)
        gabf_ref[...] = ga_ref[0].astype(bf16)
        dnbf_ref[...] = dn_ref[0].astype(bf16)

    @pl.when(nv_ref[i] > 0)
    def _():
        seq = hid_ref.shape[0]
        tok = tok_ref[...]
        iota = jax.lax.broadcasted_iota(jnp.int32, (BLK, seq), 1)
        onehot = (tok == iota).astype(bf16)
        x = jax.lax.dot_general(
            onehot, hid_ref[...], (((1,), (0,)), ((), ())),
            preferred_element_type=f32).astype(bf16)
        coef = sw_ref[...]
        tb_up = jax.lax.dot_general(
            x, upm_ref[0], (((1,), (0,)), ((), ())),
            preferred_element_type=f32)
        tb_gate = jax.lax.dot_general(
            x, gm_ref[0], (((1,), (0,)), ((), ())),
            preferred_element_type=f32)
        tbu16 = (coef * tb_up).astype(bf16)
        tbg16 = tb_gate.astype(bf16)
        p = uabf_ref.shape[0]
        pc_n = 4
        pchunk = p // pc_n
        for pc in range(pc_n):
            psl = pl.ds(pc * pchunk, pchunk)
            up_c = jax.lax.dot_general(
                tbu16, uabf_ref[psl, :], (((1,), (1,)), ((), ())),
                preferred_element_type=f32)
            gate_c = jax.lax.dot_general(
                tbg16, gabf_ref[psl, :], (((1,), (1,)), ((), ())),
                preferred_element_type=f32)
            g16 = gate_c.astype(bf16)
            u16 = up_c.astype(bf16)
            t = jnp.tanh(g16 * jnp.bfloat16(0.5))
            inter = g16 * (jnp.bfloat16(0.5) + jnp.bfloat16(0.5) * t) * u16
            part = jax.lax.dot_general(
                inter, dnbf_ref[:, psl], (((1,), (1,)), ((), ())),
                preferred_element_type=f32)
            if pc == 0:
                y_ref[...] = part
            else:
                y_ref[...] = y_ref[...] + part


def _ffn_grouped(hidden_bf, slot_token, slot_w, up_mixed, gate_mixed,
                 up_adapters, gate_adapters, down_projections, be, nv, first,
                 nb):
    seq, d = hidden_bf.shape
    nbb = slot_token.shape[0]
    p, r = up_adapters.shape[1:]
    grid_spec = pltpu.PrefetchScalarGridSpec(
        num_scalar_prefetch=3,
        grid=(nb,),
        in_specs=[
            pl.BlockSpec((seq, d), lambda i, be, nv, fs: (0, 0)),
            pl.BlockSpec((BLK, 1), lambda i, be, nv, fs: (i, 0)),
            pl.BlockSpec((BLK, 1), lambda i, be, nv, fs: (i, 0)),
            pl.BlockSpec((1, d, r), lambda i, be, nv, fs: (be[i], 0, 0)),
            pl.BlockSpec((1, d, r), lambda i, be, nv, fs: (be[i], 0, 0)),
            pl.BlockSpec((1, p, r), lambda i, be, nv, fs: (be[i], 0, 0)),
            pl.BlockSpec((1, p, r), lambda i, be, nv, fs: (be[i], 0, 0)),
            pl.BlockSpec((1, d, p), lambda i, be, nv, fs: (be[i], 0, 0)),
        ],
        out_specs=pl.BlockSpec((BLK, d), lambda i, be, nv, fs: (i, 0)),
        scratch_shapes=[
            pltpu.VMEM((p, r), jnp.bfloat16),
            pltpu.VMEM((p, r), jnp.bfloat16),
            pltpu.VMEM((d, p), jnp.bfloat16),
        ],
    )
    return pl.pallas_call(
        _ffn_grouped_body,
        grid_spec=grid_spec,
        out_shape=jax.ShapeDtypeStruct((nbb, d), jnp.float32),
        compiler_params=pltpu.CompilerParams(
            vmem_limit_bytes=64 * 1024 * 1024),
    )(be, nv, first, hidden_bf, slot_token.reshape(nbb, 1),
      slot_w.reshape(nbb, 1), up_mixed, gate_mixed,
      up_adapters, gate_adapters, down_projections)


def kernel(hidden_states, expert_indices, expert_weights, up_adapters,
           gate_adapters, up_mixture_logits, gate_mixture_logits,
           down_projections, up_bank, gate_bank):
    seq, d = hidden_states.shape
    g = seq * TOPK
    nb = g // BLK + E
    nbb = nb * BLK

    up_mixed, gate_mixed = _mix_banks(up_mixture_logits, gate_mixture_logits,
                                      up_bank, gate_bank)
    flat_w, slot, be, nv, first = _routing_metadata(
        expert_indices, expert_weights, nb)

    flat_t = jnp.arange(g, dtype=jnp.int32) // TOPK
    slot_token, slot_w = _dispatch_sc(slot, flat_t, flat_w, nbb)

    y_sorted = _ffn_grouped(hidden_states.astype(jnp.bfloat16), slot_token,
                            slot_w, up_mixed, gate_mixed, up_adapters,
                            gate_adapters, down_projections, be, nv, first,
                            nb)

    pos = slot.reshape(seq, TOPK)
    tpr = 32
    pos0 = pos[:, 0].reshape(seq // tpr, tpr)
    pos1 = pos[:, 1].reshape(seq // tpr, tpr)
    return _combine_sc(pos0, pos1, y_sorted)
